# baseline (speedup 1.0000x reference)
import jax
import jax.numpy as jnp
from jax import lax
from jax.experimental import pallas as pl
from jax.experimental.pallas import tpu as pltpu
from jax.experimental.pallas import tpu_sc as plsc

NC = 2
NS = 16
L = 16
NW = NC * NS
BATCH = 16384
EMBED = 16
NROWS = 1000000
B_PER_W = BATCH // NW
CHUNK = 128
N_CHUNKS = B_PER_W // CHUNK
N_GROUPS = B_PER_W // L
POS_W = 1.0 + 0.6931471805599453


def _sc_body(u_idx, p_idx, n_idx, utab, itab, out, idx_v, cu, cp, cn,
             part_v, sem):
    c = lax.axis_index("c")
    s = lax.axis_index("s")
    wid = s * NC + c

    base_row = wid * N_CHUNKS
    pltpu.sync_copy(u_idx.at[pl.ds(base_row, N_CHUNKS)], idx_v.at[0])
    pltpu.sync_copy(p_idx.at[pl.ds(base_row, N_CHUNKS)], idx_v.at[1])
    pltpu.sync_copy(n_idx.at[pl.ds(base_row, N_CHUNKS)], idx_v.at[2])

    copies = []
    for d in range(EMBED):
        row = pl.ds(d * NROWS, NROWS)
        for t in range(N_CHUNKS):
            dst = pl.ds(t * CHUNK, CHUNK)
            copies.append(pltpu.async_copy(
                utab.at[row].at[idx_v.at[0, t]], cu.at[d, dst], sem))
            copies.append(pltpu.async_copy(
                itab.at[row].at[idx_v.at[1, t]], cp.at[d, dst], sem))
            copies.append(pltpu.async_copy(
                itab.at[row].at[idx_v.at[2, t]], cn.at[d, dst], sem))
    for cpy in copies:
        cpy.wait()

    def group(g, acc):
        sl = pl.ds(g * L, L)
        accp = jnp.zeros((L,), jnp.float32)
        accn = jnp.zeros((L,), jnp.float32)
        for d in range(EMBED):
            gu = cu[d, sl]
            accp = accp + gu * cp[d, sl]
            accn = accn + gu * cn[d, sl]
        sp = 1.0 / (1.0 + jnp.exp(-accp))
        sn = 1.0 / (1.0 + jnp.exp(-accn))
        dp = sp - 1.0
        return acc + (POS_W * (dp * dp) + sn * sn)

    part = lax.fori_loop(0, N_GROUPS, group, jnp.zeros((L,), jnp.float32))
    part_v[...] = part
    pltpu.sync_copy(part_v, out.at[wid])


_sc_call = pl.kernel(
    _sc_body,
    out_type=jax.ShapeDtypeStruct((NW, L), jnp.float32),
    mesh=plsc.VectorSubcoreMesh(core_axis_name="c", subcore_axis_name="s"),
    scratch_types=[
        pltpu.VMEM((3, N_CHUNKS, CHUNK), jnp.int32),
        pltpu.VMEM((EMBED, B_PER_W), jnp.float32),
        pltpu.VMEM((EMBED, B_PER_W), jnp.float32),
        pltpu.VMEM((EMBED, B_PER_W), jnp.float32),
        pltpu.VMEM((L,), jnp.float32),
        pltpu.SemaphoreType.DMA,
    ],
    compiler_params=pltpu.CompilerParams(needs_layout_passes=False,
                                         use_tc_tiling_on_sc=False),
)


def _reduce_body(x_ref, o_ref):
    o_ref[0, 0] = jnp.sum(x_ref[...]) * (1.0 / (2.0 * BATCH))


_reduce_call = pl.pallas_call(
    _reduce_body,
    out_shape=jax.ShapeDtypeStruct((1, 1), jnp.float32),
    out_specs=pl.BlockSpec(memory_space=pltpu.SMEM),
)


def kernel(users, positive_items, negative_items, user_embedding,
           item_embedding):
    u2 = users.astype(jnp.int32).reshape(NW * N_CHUNKS, CHUNK)
    p2 = positive_items.astype(jnp.int32).reshape(NW * N_CHUNKS, CHUNK)
    n2 = negative_items.astype(jnp.int32).reshape(NW * N_CHUNKS, CHUNK)
    ut = user_embedding.T.reshape(EMBED * NROWS)
    it = item_embedding.T.reshape(EMBED * NROWS)
    partials = _sc_call(u2, p2, n2, ut, it)
    return _reduce_call(partials)[0, 0]


# speedup vs baseline: 3.2057x; 3.2057x over previous
import jax
import jax.numpy as jnp
from jax import lax
from jax.experimental import pallas as pl
from jax.experimental.pallas import tpu as pltpu
from jax.experimental.pallas import tpu_sc as plsc

NC = 2
NS = 16
L = 16
NW = NC * NS
BATCH = 16384
EMBED = 16
B_PER_W = BATCH // NW
CHUNK = 128
N_CHUNKS = B_PER_W // CHUNK
N_GROUPS = B_PER_W // L
POS_W = 1.0 + 0.6931471805599453


def _sc_body(u_idx, p_idx, n_idx, utab, itab, out, idx_v, rows_u, rows_p,
             rows_n, part_v, sem):
    c = lax.axis_index("c")
    s = lax.axis_index("s")
    wid = s * NC + c

    base_row = wid * N_CHUNKS
    pltpu.sync_copy(u_idx.at[pl.ds(base_row, N_CHUNKS)], idx_v.at[0])
    pltpu.sync_copy(p_idx.at[pl.ds(base_row, N_CHUNKS)], idx_v.at[1])
    pltpu.sync_copy(n_idx.at[pl.ds(base_row, N_CHUNKS)], idx_v.at[2])

    copies = []
    for t in range(N_CHUNKS):
        dst = pl.ds(t * CHUNK, CHUNK)
        copies.append(pltpu.async_copy(utab.at[idx_v.at[0, t]],
                                       rows_u.at[dst], sem))
        copies.append(pltpu.async_copy(itab.at[idx_v.at[1, t]],
                                       rows_p.at[dst], sem))
        copies.append(pltpu.async_copy(itab.at[idx_v.at[2, t]],
                                       rows_n.at[dst], sem))
    for cp in copies:
        cp.wait()

    iota = lax.iota(jnp.int32, L)
    cols = [lax.rem(iota + t, L) for t in range(L)]

    def group(g, acc):
        row = g * L + iota
        accp = jnp.zeros((L,), jnp.float32)
        accn = jnp.zeros((L,), jnp.float32)
        for t in range(L):
            gu = plsc.load_gather(rows_u, [row, cols[t]])
            gp = plsc.load_gather(rows_p, [row, cols[t]])
            gn = plsc.load_gather(rows_n, [row, cols[t]])
            accp = accp + gu * gp
            accn = accn + gu * gn
        sp = 1.0 / (1.0 + jnp.exp(-accp))
        sn = 1.0 / (1.0 + jnp.exp(-accn))
        dp = sp - 1.0
        return acc + (POS_W * (dp * dp) + sn * sn)

    part = lax.fori_loop(0, N_GROUPS, group, jnp.zeros((L,), jnp.float32))
    part_v[...] = part
    pltpu.sync_copy(part_v, out.at[wid])


_sc_call = pl.kernel(
    _sc_body,
    out_type=jax.ShapeDtypeStruct((NW, L), jnp.float32),
    mesh=plsc.VectorSubcoreMesh(core_axis_name="c", subcore_axis_name="s"),
    scratch_types=[
        pltpu.VMEM((3, N_CHUNKS, CHUNK), jnp.int32),
        pltpu.VMEM((B_PER_W, EMBED), jnp.float32),
        pltpu.VMEM((B_PER_W, EMBED), jnp.float32),
        pltpu.VMEM((B_PER_W, EMBED), jnp.float32),
        pltpu.VMEM((L,), jnp.float32),
        pltpu.SemaphoreType.DMA,
    ],
    compiler_params=pltpu.CompilerParams(needs_layout_passes=False,
                                         use_tc_tiling_on_sc=False),
)


def _reduce_body(x_ref, o_ref):
    o_ref[0, 0] = jnp.sum(x_ref[...]) * (1.0 / (2.0 * BATCH))


_reduce_call = pl.pallas_call(
    _reduce_body,
    out_shape=jax.ShapeDtypeStruct((1, 1), jnp.float32),
    out_specs=pl.BlockSpec(memory_space=pltpu.SMEM),
)


def kernel(users, positive_items, negative_items, user_embedding,
           item_embedding):
    u2 = users.astype(jnp.int32).reshape(NW * N_CHUNKS, CHUNK)
    p2 = positive_items.astype(jnp.int32).reshape(NW * N_CHUNKS, CHUNK)
    n2 = negative_items.astype(jnp.int32).reshape(NW * N_CHUNKS, CHUNK)
    partials = _sc_call(u2, p2, n2, user_embedding, item_embedding)
    return _reduce_call(partials)[0, 0]


# speedup vs baseline: 12.5721x; 3.9218x over previous
import jax
import jax.numpy as jnp
from jax import lax
from jax.experimental import pallas as pl
from jax.experimental.pallas import tpu as pltpu
from jax.experimental.pallas import tpu_sc as plsc

NC = 2
NS = 16
L = 16
NW = NC * NS
BATCH = 16384
EMBED = 16
NROWS = 1000000
PSTRIDE = 1000064
LINSZ = EMBED * PSTRIDE
TAIL = NROWS % 128
TAIL0 = NROWS - TAIL
WCH = 2048
NFULL = NROWS // WCH
B_PER_W = BATCH // NW
CHUNK = 128
N_CHUNKS = B_PER_W // CHUNK
N_GROUPS = B_PER_W // L
POS_W = 1.0 + 0.6931471805599453


def _detile_body(ut, it, tu, ti, ulin, ilin, buf, tbuf, sem, sem2):
    wid = lax.axis_index("s") * NC + lax.axis_index("c")
    tab_sel = wid % 2
    R = (wid // 2) % 2
    widC = wid // 4

    def go(tab, lin):
        def step(jj, carry):
            off = (widC + 8 * jj) * WCH
            pltpu.async_copy(tab.at[pl.ds(R * 8, 8), pl.ds(off, WCH)],
                             buf, sem).wait()
            wr = []
            for cl in range(WCH // 128):
                for ds_ in range(8):
                    d = R * 8 + ds_
                    wr.append(pltpu.async_copy(
                        buf.at[ds_, pl.ds(cl * 128, 128)],
                        lin.at[pl.ds(d * PSTRIDE + off + cl * 128, 128)],
                        sem2))
            for w in wr:
                w.wait()
            return carry

        lax.fori_loop(0, NFULL // 8, step, 0)

        @pl.when(widC == 0)
        def _():
            off = NFULL * WCH
            pltpu.async_copy(tab.at[pl.ds(R * 8, 8), pl.ds(off, 512)],
                             tbuf, sem).wait()
            wr = []
            for cl in range(4):
                for ds_ in range(8):
                    d = R * 8 + ds_
                    wr.append(pltpu.async_copy(
                        tbuf.at[ds_, pl.ds(cl * 128, 128)],
                        lin.at[pl.ds(d * PSTRIDE + off + cl * 128, 128)],
                        sem2))
            for w in wr:
                w.wait()

    @pl.when(tab_sel == 0)
    def _():
        go(ut, ulin)

    @pl.when(tab_sel == 1)
    def _():
        go(it, ilin)

    @pl.when(wid == 0)
    def _():
        pltpu.sync_copy(tu, buf.at[0, pl.ds(0, EMBED * TAIL)])
        for d in range(EMBED):
            pltpu.sync_copy(buf.at[0, pl.ds(d * TAIL, TAIL)],
                            ulin.at[pl.ds(d * PSTRIDE + TAIL0, TAIL)])

    @pl.when(wid == 1)
    def _():
        pltpu.sync_copy(ti, buf.at[0, pl.ds(0, EMBED * TAIL)])
        for d in range(EMBED):
            pltpu.sync_copy(buf.at[0, pl.ds(d * TAIL, TAIL)],
                            ilin.at[pl.ds(d * PSTRIDE + TAIL0, TAIL)])


_detile_call = pl.kernel(
    _detile_body,
    out_type=[jax.ShapeDtypeStruct((LINSZ,), jnp.float32),
              jax.ShapeDtypeStruct((LINSZ,), jnp.float32)],
    mesh=plsc.VectorSubcoreMesh(core_axis_name="c", subcore_axis_name="s"),
    scratch_types=[
        pltpu.VMEM((8, WCH), jnp.float32),
        pltpu.VMEM((8, 512), jnp.float32),
        pltpu.SemaphoreType.DMA,
        pltpu.SemaphoreType.DMA,
    ],
    compiler_params=pltpu.CompilerParams(needs_layout_passes=False),
)


def _sc_body(u_idx, p_idx, n_idx, utab, itab, out, idx_v, cu, cp, cn,
             part_v, sem):
    c = lax.axis_index("c")
    s = lax.axis_index("s")
    wid = s * NC + c

    base_row = wid * N_CHUNKS
    pltpu.sync_copy(u_idx.at[pl.ds(base_row, N_CHUNKS)], idx_v.at[0])
    pltpu.sync_copy(p_idx.at[pl.ds(base_row, N_CHUNKS)], idx_v.at[1])
    pltpu.sync_copy(n_idx.at[pl.ds(base_row, N_CHUNKS)], idx_v.at[2])

    copies = []
    for d in range(EMBED):
        row = pl.ds(d * PSTRIDE, NROWS)
        for t in range(N_CHUNKS):
            dst = pl.ds(t * CHUNK, CHUNK)
            copies.append(pltpu.async_copy(
                utab.at[row].at[idx_v.at[0, t]], cu.at[d, dst], sem))
            copies.append(pltpu.async_copy(
                itab.at[row].at[idx_v.at[1, t]], cp.at[d, dst], sem))
            copies.append(pltpu.async_copy(
                itab.at[row].at[idx_v.at[2, t]], cn.at[d, dst], sem))
    for cpy in copies:
        cpy.wait()

    def group(g, acc):
        sl = pl.ds(g * L, L)
        accp = jnp.zeros((L,), jnp.float32)
        accn = jnp.zeros((L,), jnp.float32)
        for d in range(EMBED):
            gu = cu[d, sl]
            accp = accp + gu * cp[d, sl]
            accn = accn + gu * cn[d, sl]
        sp = 1.0 / (1.0 + jnp.exp(-accp))
        sn = 1.0 / (1.0 + jnp.exp(-accn))
        dp = sp - 1.0
        return acc + (POS_W * (dp * dp) + sn * sn)

    part = lax.fori_loop(0, N_GROUPS, group, jnp.zeros((L,), jnp.float32))
    part_v[...] = part
    pltpu.sync_copy(part_v, out.at[wid])


_sc_call = pl.kernel(
    _sc_body,
    out_type=jax.ShapeDtypeStruct((NW, L), jnp.float32),
    mesh=plsc.VectorSubcoreMesh(core_axis_name="c", subcore_axis_name="s"),
    scratch_types=[
        pltpu.VMEM((3, N_CHUNKS, CHUNK), jnp.int32),
        pltpu.VMEM((EMBED, B_PER_W), jnp.float32),
        pltpu.VMEM((EMBED, B_PER_W), jnp.float32),
        pltpu.VMEM((EMBED, B_PER_W), jnp.float32),
        pltpu.VMEM((L,), jnp.float32),
        pltpu.SemaphoreType.DMA,
    ],
    compiler_params=pltpu.CompilerParams(needs_layout_passes=False,
                                         use_tc_tiling_on_sc=False),
)


def _reduce_body(x_ref, o_ref):
    o_ref[0, 0] = jnp.sum(x_ref[...]) * (1.0 / (2.0 * BATCH))


_reduce_call = pl.pallas_call(
    _reduce_body,
    out_shape=jax.ShapeDtypeStruct((1, 1), jnp.float32),
    out_specs=pl.BlockSpec(memory_space=pltpu.SMEM),
)


def kernel(users, positive_items, negative_items, user_embedding,
           item_embedding):
    u2 = users.astype(jnp.int32).reshape(NW * N_CHUNKS, CHUNK)
    p2 = positive_items.astype(jnp.int32).reshape(NW * N_CHUNKS, CHUNK)
    n2 = negative_items.astype(jnp.int32).reshape(NW * N_CHUNKS, CHUNK)
    ut = user_embedding.T
    it = item_embedding.T
    tu = user_embedding[TAIL0:].T.reshape(EMBED * TAIL)
    ti = item_embedding[TAIL0:].T.reshape(EMBED * TAIL)
    u_lin, i_lin = _detile_call(ut, it, tu, ti)
    partials = _sc_call(u2, p2, n2, u_lin, i_lin)
    return _reduce_call(partials)[0, 0]
